# baseline (speedup 1.0000x reference)
import jax
import jax.numpy as jnp
from jax.experimental import pallas as pl
from jax.experimental.pallas import tpu as pltpu


def _conf_kernel(pred_ref, tgt_ref, hist_ref, keys_ref, vals_ref, acc_ref):
    i = pl.program_id(0)
    nb = pl.num_programs(0)
    C = pred_ref.shape[1]
    B = pred_ref.shape[0]

    @pl.when(i == 0)
    def _init():
        acc_ref[...] = jnp.zeros_like(acc_ref)

    p = pred_ref[...]
    t = tgt_ref[0]
    col = jax.lax.broadcasted_iota(jnp.int32, (B, C), 1)
    m = jnp.max(p, axis=1, keepdims=True)
    label = jnp.min(jnp.where(p == m, col, C), axis=1, keepdims=True)
    oh_l = (col == label).astype(jnp.float32)
    cls = jax.lax.broadcasted_iota(jnp.int32, (C, B), 0)
    oh_t = (cls == t).astype(jnp.float32)
    acc_ref[...] += jax.lax.dot_general(
        oh_t, oh_l, (((1,), (0,)), ((), ())),
        preferred_element_type=jnp.float32)

    @pl.when(i == nb - 1)
    def _finish():
        h = acc_ref[...]
        hist_ref[...] = h.astype(jnp.int32)
        r = jax.lax.broadcasted_iota(jnp.int32, (C, C), 0)
        c = jax.lax.broadcasted_iota(jnp.int32, (C, C), 1)
        eye = (r == c)
        dcol = jnp.sum(jnp.where(eye, h, 0.0), axis=1, keepdims=True)
        drow = jnp.sum(jnp.where(eye, h, 0.0), axis=0, keepdims=True)
        wins = (drow < dcol) | ((drow == dcol) & (c < r))
        rank = jnp.sum(wins.astype(jnp.float32), axis=1, keepdims=True)
        q = (rank == c.astype(jnp.float32)).astype(jnp.float32)
        vals_ref[...] = jnp.sum(q * dcol, axis=0, keepdims=True).astype(jnp.int32)
        keys_ref[...] = jnp.sum(q * r.astype(jnp.float32), axis=0,
                                keepdims=True).astype(jnp.int32)


def _pick_block(n):
    for b in (8000, 8192, 10000, 5000, 4096, 4000, 2048, 2000, 1024, 1000,
              512, 500, 256, 200, 128, 100, 64, 40, 32, 16, 8):
        if n % b == 0:
            return b
    return None


def kernel(pred, target):
    n, n_class = pred.shape
    b = _pick_block(n)
    if b is None:
        b = 512
        npad = (n + b - 1) // b * b
        pred = jnp.pad(pred, ((0, npad - n), (0, 0)))
        target = jnp.pad(target, (0, npad - n), constant_values=-1)
        n = npad
    nb = n // b
    tgt3 = target.astype(jnp.int32).reshape(nb, 1, b)
    hist, keys, vals = pl.pallas_call(
        _conf_kernel,
        grid=(nb,),
        in_specs=[
            pl.BlockSpec((b, n_class), lambda i: (i, 0)),
            pl.BlockSpec((1, 1, b), lambda i: (i, 0, 0)),
        ],
        out_specs=[
            pl.BlockSpec((n_class, n_class), lambda i: (0, 0)),
            pl.BlockSpec((1, n_class), lambda i: (0, 0)),
            pl.BlockSpec((1, n_class), lambda i: (0, 0)),
        ],
        out_shape=[
            jax.ShapeDtypeStruct((n_class, n_class), jnp.int32),
            jax.ShapeDtypeStruct((1, n_class), jnp.int32),
            jax.ShapeDtypeStruct((1, n_class), jnp.int32),
        ],
        scratch_shapes=[pltpu.VMEM((n_class, n_class), jnp.float32)],
        compiler_params=pltpu.CompilerParams(
            dimension_semantics=("arbitrary",)),
    )(pred, tgt3)
    return hist, keys.reshape(n_class), vals.reshape(n_class)


# speedup vs baseline: 1.2338x; 1.2338x over previous
import jax
import jax.numpy as jnp
from jax.experimental import pallas as pl
from jax.experimental.pallas import tpu as pltpu


def _conf_kernel(pred_ref, tgt_ref, hist_ref, keys_ref, vals_ref,
                 acc_ref, col_ref, cls_ref):
    i = pl.program_id(0)
    nb = pl.num_programs(0)
    B, C = pred_ref.shape
    S = B // 8

    @pl.when(i == 0)
    def _init():
        acc_ref[...] = jnp.zeros_like(acc_ref)
        col_ref[...] = jax.lax.broadcasted_iota(
            jnp.int32, (S, C), 1).astype(jnp.float32)
        cls_ref[...] = jax.lax.broadcasted_iota(jnp.int32, (C, S), 0)

    col = col_ref[...]
    cls = cls_ref[...]
    one = jnp.float32(1.0)
    zero = jnp.float32(0.0)
    for s in range(8):
        p = pred_ref[pl.ds(s * S, S), :]
        t = tgt_ref[pl.ds(s, 1), :]
        m = jnp.max(p, axis=1, keepdims=True)
        lab = jnp.min(jnp.where(p == m, col, float(C)),
                      axis=1, keepdims=True)
        oh_l = jnp.where(col == lab, one, zero)
        oh_t = jnp.where(cls == t, one, zero)
        acc_ref[...] += jax.lax.dot_general(
            oh_t, oh_l, (((1,), (0,)), ((), ())),
            preferred_element_type=jnp.float32)

    @pl.when(i == nb - 1)
    def _finish():
        h = acc_ref[...]
        hist_ref[...] = h.astype(jnp.int32)
        r = jax.lax.broadcasted_iota(jnp.int32, (C, C), 0).astype(jnp.float32)
        c = jax.lax.broadcasted_iota(jnp.int32, (C, C), 1).astype(jnp.float32)
        eye = (r == c)
        dcol = jnp.sum(jnp.where(eye, h, 0.0), axis=1, keepdims=True)
        drow = jnp.sum(jnp.where(eye, h, 0.0), axis=0, keepdims=True)
        wins = (drow < dcol) | ((drow == dcol) & (c < r))
        rank = jnp.sum(jnp.where(wins, 1.0, 0.0), axis=1, keepdims=True)
        q = jnp.where(rank == c, 1.0, 0.0)
        vals_ref[...] = jnp.sum(q * dcol, axis=0, keepdims=True).astype(jnp.int32)
        keys_ref[...] = jnp.sum(q * r, axis=0, keepdims=True).astype(jnp.int32)


def _pick_block(n):
    best = None
    for b in range(64, 16385, 64):
        if n % b == 0:
            best = b
    return best


def kernel(pred, target):
    n, n_class = pred.shape
    b = _pick_block(n)
    if b is None:
        b = min(512, ((n + 63) // 64) * 64)
        npad = (n + b - 1) // b * b
        pred = jnp.pad(pred, ((0, npad - n), (0, 0)))
        target = jnp.pad(target, (0, npad - n), constant_values=-1)
        n = npad
    nb = n // b
    tgt2 = target.astype(jnp.int32).reshape(8 * nb, b // 8)
    hist, keys, vals = pl.pallas_call(
        _conf_kernel,
        grid=(nb,),
        in_specs=[
            pl.BlockSpec((b, n_class), lambda i: (i, 0)),
            pl.BlockSpec((8, b // 8), lambda i: (i, 0)),
        ],
        out_specs=[
            pl.BlockSpec((n_class, n_class), lambda i: (0, 0)),
            pl.BlockSpec((1, n_class), lambda i: (0, 0)),
            pl.BlockSpec((1, n_class), lambda i: (0, 0)),
        ],
        out_shape=[
            jax.ShapeDtypeStruct((n_class, n_class), jnp.int32),
            jax.ShapeDtypeStruct((1, n_class), jnp.int32),
            jax.ShapeDtypeStruct((1, n_class), jnp.int32),
        ],
        scratch_shapes=[
            pltpu.VMEM((n_class, n_class), jnp.float32),
            pltpu.VMEM((b // 8, n_class), jnp.float32),
            pltpu.VMEM((n_class, b // 8), jnp.int32),
        ],
        compiler_params=pltpu.CompilerParams(
            dimension_semantics=("arbitrary",),
            fuse_transposed_lhs_in_matmul=True),
    )(pred, tgt2)
    return hist, keys.reshape(n_class), vals.reshape(n_class)
